# baseline (speedup 1.0000x reference)
import functools

import jax
import jax.numpy as jnp
from jax import lax
from jax.experimental import pallas as pl
from jax.experimental.pallas import tpu as pltpu
from jax.experimental.pallas import tpu_sc as plsc

B_TOT = 16384
LSEQ = 200
VOCAB = 1000
VPAD = 1024
EMB = 128
NC = 2
NS = 16
NW = NC * NS
PB = B_TOT // 2
PRW = PB // NW
CH = 16
NCHUNK = PRW // CH
CW = CH * LSEQ
CVR = CW // 128
UNROLL = 20


def _scores_body(w_ref, t_ref, o_ref):
    o_ref[...] = lax.dot_general(
        w_ref[...], t_ref[...], (((1,), (1,)), ((), ())),
        preferred_element_type=jnp.float32,
        precision=lax.Precision.HIGHEST,
    )


def _compute_scores(w8, table_pad):
    return pl.pallas_call(
        _scores_body,
        out_shape=jax.ShapeDtypeStruct((8, VPAD), jnp.float32),
    )(w8, table_pad)


def _sc_pool_body(ids_hbm, s01_hbm, out0_hbm, out1_hbm,
                  s01_v, sp_v, ids_a, ids_b, o0lo_v, o1lo_v, o0hi_v, o1hi_v,
                  sem_a, sem_b):
    wid = lax.axis_index("c") * NS + lax.axis_index("s")
    base_word = wid * (PRW * LSEQ)
    pltpu.sync_copy(s01_hbm, s01_v)
    lane_base = lax.iota(jnp.int32, 16) * LSEQ

    def _pk(i, _):
        b0 = plsc.bitcast(s01_v[pl.ds(i * 16, 16)], jnp.int32)
        b1 = plsc.bitcast(s01_v[pl.ds(VPAD + i * 16, 16)], jnp.int32)
        lo16 = lax.shift_right_logical(b0 + 0x8000, 16)
        hi16 = (b1 + 0x8000) & ~0xFFFF
        sp_v[pl.ds(i * 16, 16)] = lo16 | hi16
        return 0

    lax.fori_loop(0, VPAD // 16, _pk, 0)

    def start(g, buf, sem):
        pltpu.async_copy(ids_hbm.at[pl.ds(base_word + g * CW, CW)], buf, sem)

    def wait(buf, sem):
        pltpu.make_async_copy(ids_hbm.at[pl.ds(0, CW)], buf, sem).wait()

    def compute(g, buf):
        def body(l, carry):
            a0lo, a1lo, a0hi, a1hi = carry
            for u in range(UNROLL):
                w = plsc.load_gather(buf, [lane_base + (l * UNROLL + u)])
                ilo = w & 0xFFFF
                ihi = lax.shift_right_logical(w, 16)
                plo = plsc.load_gather(sp_v, [ilo])
                phi = plsc.load_gather(sp_v, [ihi])
                a0lo = a0lo + plsc.bitcast(lax.shift_left(plo, 16), jnp.float32)
                a1lo = a1lo + plsc.bitcast(plo & ~0xFFFF, jnp.float32)
                a0hi = a0hi + plsc.bitcast(lax.shift_left(phi, 16), jnp.float32)
                a1hi = a1hi + plsc.bitcast(phi & ~0xFFFF, jnp.float32)
            return (a0lo, a1lo, a0hi, a1hi)

        zero = jnp.zeros((16,), jnp.float32)
        a0lo, a1lo, a0hi, a1hi = lax.fori_loop(
            0, LSEQ // UNROLL, body, (zero, zero, zero, zero))
        o0lo_v[pl.ds(g * CH, CH)] = a0lo
        o1lo_v[pl.ds(g * CH, CH)] = a1lo
        o0hi_v[pl.ds(g * CH, CH)] = a0hi
        o1hi_v[pl.ds(g * CH, CH)] = a1hi

    start(0, ids_a, sem_a)

    @pl.loop(0, NCHUNK // 2)
    def _pair(h):
        g0 = 2 * h
        start(g0 + 1, ids_b, sem_b)
        wait(ids_a, sem_a)
        compute(g0, ids_a)
        start(jnp.minimum(g0 + 2, NCHUNK - 1), ids_a, sem_a)
        wait(ids_b, sem_b)
        compute(g0 + 1, ids_b)

    wait(ids_a, sem_a)
    lo = wid * PRW
    pltpu.sync_copy(o0lo_v, out0_hbm.at[pl.ds(lo, PRW)])
    pltpu.sync_copy(o1lo_v, out1_hbm.at[pl.ds(lo, PRW)])
    pltpu.sync_copy(o0hi_v, out0_hbm.at[pl.ds(PB + lo, PRW)])
    pltpu.sync_copy(o1hi_v, out1_hbm.at[pl.ds(PB + lo, PRW)])


@functools.cache
def _build_sc_pool():
    return pl.kernel(
        _sc_pool_body,
        out_type=(jax.ShapeDtypeStruct((B_TOT,), jnp.float32),
                  jax.ShapeDtypeStruct((B_TOT,), jnp.float32)),
        mesh=plsc.VectorSubcoreMesh(core_axis_name="c", subcore_axis_name="s",
                                    num_cores=NC, num_subcores=NS),
        compiler_params=pltpu.CompilerParams(needs_layout_passes=False,
                                             use_tc_tiling_on_sc=False),
        scratch_types=[
            pltpu.VMEM((2 * VPAD,), jnp.float32),
            pltpu.VMEM((VPAD,), jnp.int32),
            pltpu.VMEM((CW,), jnp.int32),
            pltpu.VMEM((CW,), jnp.int32),
            pltpu.VMEM((PRW,), jnp.float32),
            pltpu.VMEM((PRW,), jnp.float32),
            pltpu.VMEM((PRW,), jnp.float32),
            pltpu.VMEM((PRW,), jnp.float32),
            pltpu.SemaphoreType.DMA,
            pltpu.SemaphoreType.DMA,
        ],
    )


def _lsm_body(s0_ref, s1_ref, b_ref, o0_ref, o1_ref):
    l0 = s0_ref[...] * (1.0 / LSEQ) + b_ref[0]
    l1 = s1_ref[...] * (1.0 / LSEQ) + b_ref[1]
    m = jnp.maximum(l0, l1)
    lse = m + jnp.log(jnp.exp(l0 - m) + jnp.exp(l1 - m))
    o0_ref[...] = l0 - lse
    o1_ref[...] = l1 - lse


def _log_softmax(sums0, sums1, b):
    return pl.pallas_call(
        _lsm_body,
        in_specs=[
            pl.BlockSpec(memory_space=pltpu.VMEM),
            pl.BlockSpec(memory_space=pltpu.VMEM),
            pl.BlockSpec(memory_space=pltpu.SMEM),
        ],
        out_specs=(pl.BlockSpec(memory_space=pltpu.VMEM),
                   pl.BlockSpec(memory_space=pltpu.VMEM)),
        out_shape=(jax.ShapeDtypeStruct((128, 128), jnp.float32),
                   jax.ShapeDtypeStruct((128, 128), jnp.float32)),
    )(sums0.reshape(128, 128), sums1.reshape(128, 128), b)


def kernel(ids, table, W, b):
    ids32 = ids.astype(jnp.int32)
    ids_pairs = (ids32[:PB] | (ids32[PB:] << 16)).reshape(-1)
    table_pad = jnp.zeros((VPAD, EMB), jnp.float32).at[:VOCAB].set(table)
    w8 = jnp.zeros((8, EMB), jnp.float32).at[:2].set(W)
    s8 = _compute_scores(w8, table_pad)
    s01 = s8[:2].reshape(-1)
    sums0, sums1 = _build_sc_pool()(ids_pairs, s01)
    o0, o1 = _log_softmax(sums0, sums1, b)
    return jnp.stack([o0.reshape(-1), o1.reshape(-1)], axis=-1)


# speedup vs baseline: 1.0287x; 1.0287x over previous
import functools

import jax
import jax.numpy as jnp
from jax import lax
from jax.experimental import pallas as pl
from jax.experimental.pallas import tpu as pltpu
from jax.experimental.pallas import tpu_sc as plsc

B_TOT = 16384
LSEQ = 200
VOCAB = 1000
VPAD = 1024
EMB = 128
NC = 2
NS = 16
NW = NC * NS
PB = B_TOT // 2
PRW = PB // NW
CH = 16
NCHUNK = PRW // CH
CW = CH * LSEQ
CVR = CW // 128
UNROLL = 8


def _scores_body(w_ref, t_ref, o_ref):
    o_ref[...] = lax.dot_general(
        w_ref[...], t_ref[...], (((1,), (1,)), ((), ())),
        preferred_element_type=jnp.float32,
        precision=lax.Precision.HIGHEST,
    )


def _compute_scores(w8, table_pad):
    return pl.pallas_call(
        _scores_body,
        out_shape=jax.ShapeDtypeStruct((8, VPAD), jnp.float32),
    )(w8, table_pad)


def _sc_pool_body(ids_hbm, s01_hbm, out0_hbm, out1_hbm,
                  s01_v, sp_v, ids_a, ids_b, o0lo_v, o1lo_v, o0hi_v, o1hi_v,
                  sem_a, sem_b):
    wid = lax.axis_index("c") * NS + lax.axis_index("s")
    base_word = wid * (PRW * LSEQ)
    pltpu.sync_copy(s01_hbm, s01_v)
    lane_base = lax.iota(jnp.int32, 16) * LSEQ

    def _pk(i, _):
        b0 = plsc.bitcast(s01_v[pl.ds(i * 16, 16)], jnp.int32)
        b1 = plsc.bitcast(s01_v[pl.ds(VPAD + i * 16, 16)], jnp.int32)
        lo16 = lax.shift_right_logical(b0 + 0x8000, 16)
        hi16 = (b1 + 0x8000) & ~0xFFFF
        sp_v[pl.ds(i * 16, 16)] = lo16 | hi16
        return 0

    lax.fori_loop(0, VPAD // 16, _pk, 0)

    def start(g, buf, sem):
        pltpu.async_copy(ids_hbm.at[pl.ds(base_word + g * CW, CW)], buf, sem)

    def wait(buf, sem):
        pltpu.make_async_copy(ids_hbm.at[pl.ds(0, CW)], buf, sem).wait()

    def compute(g, buf):
        def body(l, carry):
            a0lo, a1lo, a0hi, a1hi = carry
            for u in range(UNROLL):
                w = plsc.load_gather(buf, [lane_base + (l * UNROLL + u)])
                ilo = w & 0xFFFF
                ihi = lax.shift_right_logical(w, 16)
                plo = plsc.load_gather(sp_v, [ilo])
                phi = plsc.load_gather(sp_v, [ihi])
                a0lo = a0lo + plsc.bitcast(lax.shift_left(plo, 16), jnp.float32)
                a1lo = a1lo + plsc.bitcast(plo & ~0xFFFF, jnp.float32)
                a0hi = a0hi + plsc.bitcast(lax.shift_left(phi, 16), jnp.float32)
                a1hi = a1hi + plsc.bitcast(phi & ~0xFFFF, jnp.float32)
            return (a0lo, a1lo, a0hi, a1hi)

        zero = jnp.zeros((16,), jnp.float32)
        a0lo, a1lo, a0hi, a1hi = lax.fori_loop(
            0, LSEQ // UNROLL, body, (zero, zero, zero, zero))
        o0lo_v[pl.ds(g * CH, CH)] = a0lo
        o1lo_v[pl.ds(g * CH, CH)] = a1lo
        o0hi_v[pl.ds(g * CH, CH)] = a0hi
        o1hi_v[pl.ds(g * CH, CH)] = a1hi

    start(0, ids_a, sem_a)

    @pl.loop(0, NCHUNK // 2)
    def _pair(h):
        g0 = 2 * h
        start(g0 + 1, ids_b, sem_b)
        wait(ids_a, sem_a)
        compute(g0, ids_a)
        start(jnp.minimum(g0 + 2, NCHUNK - 1), ids_a, sem_a)
        wait(ids_b, sem_b)
        compute(g0 + 1, ids_b)

    wait(ids_a, sem_a)
    lo = wid * PRW
    pltpu.sync_copy(o0lo_v, out0_hbm.at[pl.ds(lo, PRW)])
    pltpu.sync_copy(o1lo_v, out1_hbm.at[pl.ds(lo, PRW)])
    pltpu.sync_copy(o0hi_v, out0_hbm.at[pl.ds(PB + lo, PRW)])
    pltpu.sync_copy(o1hi_v, out1_hbm.at[pl.ds(PB + lo, PRW)])


@functools.cache
def _build_sc_pool():
    return pl.kernel(
        _sc_pool_body,
        out_type=(jax.ShapeDtypeStruct((B_TOT,), jnp.float32),
                  jax.ShapeDtypeStruct((B_TOT,), jnp.float32)),
        mesh=plsc.VectorSubcoreMesh(core_axis_name="c", subcore_axis_name="s",
                                    num_cores=NC, num_subcores=NS),
        compiler_params=pltpu.CompilerParams(needs_layout_passes=False,
                                             use_tc_tiling_on_sc=False),
        scratch_types=[
            pltpu.VMEM((2 * VPAD,), jnp.float32),
            pltpu.VMEM((VPAD,), jnp.int32),
            pltpu.VMEM((CW,), jnp.int32),
            pltpu.VMEM((CW,), jnp.int32),
            pltpu.VMEM((PRW,), jnp.float32),
            pltpu.VMEM((PRW,), jnp.float32),
            pltpu.VMEM((PRW,), jnp.float32),
            pltpu.VMEM((PRW,), jnp.float32),
            pltpu.SemaphoreType.DMA,
            pltpu.SemaphoreType.DMA,
        ],
    )


def _lsm_body(s0_ref, s1_ref, b_ref, o0_ref, o1_ref):
    l0 = s0_ref[...] * (1.0 / LSEQ) + b_ref[0]
    l1 = s1_ref[...] * (1.0 / LSEQ) + b_ref[1]
    m = jnp.maximum(l0, l1)
    lse = m + jnp.log(jnp.exp(l0 - m) + jnp.exp(l1 - m))
    o0_ref[...] = l0 - lse
    o1_ref[...] = l1 - lse


def _log_softmax(sums0, sums1, b):
    return pl.pallas_call(
        _lsm_body,
        in_specs=[
            pl.BlockSpec(memory_space=pltpu.VMEM),
            pl.BlockSpec(memory_space=pltpu.VMEM),
            pl.BlockSpec(memory_space=pltpu.SMEM),
        ],
        out_specs=(pl.BlockSpec(memory_space=pltpu.VMEM),
                   pl.BlockSpec(memory_space=pltpu.VMEM)),
        out_shape=(jax.ShapeDtypeStruct((128, 128), jnp.float32),
                   jax.ShapeDtypeStruct((128, 128), jnp.float32)),
    )(sums0.reshape(128, 128), sums1.reshape(128, 128), b)


def kernel(ids, table, W, b):
    ids32 = ids.astype(jnp.int32)
    ids_pairs = (ids32[:PB] | (ids32[PB:] << 16)).reshape(-1)
    table_pad = jnp.zeros((VPAD, EMB), jnp.float32).at[:VOCAB].set(table)
    w8 = jnp.zeros((8, EMB), jnp.float32).at[:2].set(W)
    s8 = _compute_scores(w8, table_pad)
    s01 = s8[:2].reshape(-1)
    sums0, sums1 = _build_sc_pool()(ids_pairs, s01)
    o0, o1 = _log_softmax(sums0, sums1, b)
    return jnp.stack([o0.reshape(-1), o1.reshape(-1)], axis=-1)
